# baseline (speedup 1.0000x reference)
import jax
import jax.numpy as jnp
from jax import lax
from jax.experimental import pallas as pl
from jax.experimental.pallas import tpu as pltpu, tpu_sc as plsc

B, S, DEPTH = 1024, 50, 1000
N = B * S
NW = 32
PER_W = N // NW
BPW = B // NW
CB = 2
CH = CB * S
NCHUNK = BPW // CB
GROUPS = (CH + 15) // 16
IDX_PAD = 1616


def _sc_body(idx_hbm, zeros_hbm, out_hbm, idx_v, buf):
    wid = lax.axis_index("s") * 2 + lax.axis_index("c")
    base = wid * PER_W
    b0 = wid * BPW
    pltpu.sync_copy(idx_hbm.at[pl.ds(base, PER_W)], idx_v.at[pl.ds(0, PER_W)])
    pltpu.sync_copy(zeros_hbm, buf)
    iota = lax.iota(jnp.int32, 16)
    ones = jnp.full((16,), 1.0, jnp.float32)
    zeros = jnp.zeros((16,), jnp.float32)

    def scatter(c, val):
        for j in range(GROUPS):
            r = iota + j * 16
            mask = r < CH
            bl = r // S
            sl = r % S
            cols = idx_v[pl.ds(c * CH + j * 16, 16)]
            plsc.store_scatter(buf, [bl, sl, cols], val, mask=mask)

    def chunk(c, carry):
        scatter(c, ones)
        pltpu.sync_copy(buf, out_hbm.at[pl.ds(b0 + c * CB, CB)])
        scatter(c, zeros)
        return carry

    lax.fori_loop(0, NCHUNK, chunk, 0)


def kernel(inputs):
    idx = inputs.astype(jnp.int32).reshape(N)
    zblock = jnp.zeros((CB, S, DEPTH), jnp.float32)
    mesh = plsc.VectorSubcoreMesh(core_axis_name="c", subcore_axis_name="s")
    k = pl.kernel(
        _sc_body,
        out_type=jax.ShapeDtypeStruct((B, S, DEPTH), jnp.float32),
        mesh=mesh,
        compiler_params=pltpu.CompilerParams(use_tc_tiling_on_sc=False, needs_layout_passes=False),
        scratch_types=[
            pltpu.VMEM((IDX_PAD,), jnp.int32),
            pltpu.VMEM((CB, S, DEPTH), jnp.float32),
        ],
    )
    return k(idx, zblock)


# speedup vs baseline: 9.3129x; 9.3129x over previous
import jax
import jax.numpy as jnp
from jax import lax
from jax.experimental import pallas as pl

B, S, DEPTH = 1024, 50, 1000


def _onehot_t_body(idx_ref, out_ref):
    row = idx_ref[0, 0, :]
    d_iota = lax.broadcasted_iota(jnp.int32, (DEPTH, B), 0)
    out_ref[0] = (row[None, :] == d_iota).astype(jnp.float32)


def kernel(inputs):
    idx_t = inputs.astype(jnp.int32).T.reshape(S, 1, B)
    out_t = pl.pallas_call(
        _onehot_t_body,
        grid=(S,),
        in_specs=[pl.BlockSpec((1, 1, B), lambda i: (i, 0, 0))],
        out_specs=pl.BlockSpec((1, DEPTH, B), lambda i: (i, 0, 0)),
        out_shape=jax.ShapeDtypeStruct((S, DEPTH, B), jnp.float32),
    )(idx_t)
    return out_t.transpose(2, 0, 1)
